# baseline (speedup 1.0000x reference)
import functools

import jax
import jax.numpy as jnp
from jax import lax
from jax.experimental import pallas as pl
from jax.experimental.pallas import tpu as pltpu
from jax.experimental.pallas import tpu_sc as plsc

NUM_NODES = 512
NODE_DIM = 64
NUM_TIMES = 288
TIME_DIM = 32
WEEKDAY_DIM = 16
MODEL_DIM = 128
B, T = 32, 12
BT = B * T
LANES = 16
DBLK = MODEL_DIM // LANES

NUM_WORKERS = 32
RPW = BT // NUM_WORKERS
HALF = NUM_NODES // 2


def _prep_body(time_ref, wd_ref, node_ref, wn_ref, tt_ref, wt_ref, wdt_ref,
               ww_ref, emb_s_ref, emb_t_ref):
    emb_s_ref[...] = jnp.dot(node_ref[...], wn_ref[...],
                             preferred_element_type=jnp.float32)
    t_idx = time_ref[...]
    w_idx = wd_ref[...]
    t_iota = lax.broadcasted_iota(jnp.int32, (BT, NUM_TIMES), 1)
    w_iota = lax.broadcasted_iota(jnp.int32, (BT, 8), 1)
    t_oh = (t_idx == t_iota).astype(jnp.float32)
    w_oh = (w_idx == w_iota).astype(jnp.float32)
    g_t = jnp.dot(t_oh, tt_ref[...], preferred_element_type=jnp.float32)
    g_w = jnp.dot(w_oh, wdt_ref[...], preferred_element_type=jnp.float32)
    emb_t_ref[...] = (
        jnp.dot(g_t, wt_ref[...], preferred_element_type=jnp.float32)
        + jnp.dot(g_w, ww_ref[...], preferred_element_type=jnp.float32))


_prep = pl.pallas_call(
    _prep_body,
    out_shape=(
        jax.ShapeDtypeStruct((NUM_NODES, MODEL_DIM), jnp.float32),
        jax.ShapeDtypeStruct((BT, MODEL_DIM), jnp.float32),
    ),
)


def _sc_writer_body(emb_s_hbm, emb_t_hbm, out_hbm, s_v, t_v, buf_v,
                    sem0, sem1):
    wid = lax.axis_index("s") * 2 + lax.axis_index("c")
    base = wid * RPW
    off = base % 8
    aligned = pl.multiple_of(base - off, 8)
    pltpu.sync_copy(emb_t_hbm.at[pl.ds(aligned, RPW + 4)], t_v)

    bufs = (buf_v.at[0], buf_v.at[1])
    sems = (sem0, sem1)

    def compute_row(r, slot):
        tvecs = [t_v[r + off, pl.ds(db * LANES, LANES)] for db in range(DBLK)]
        buf = bufs[slot]

        def n_body(n, carry):
            for db in range(DBLK):
                sl = pl.ds(db * LANES, LANES)
                buf[n, sl] = s_v[n, sl] + tvecs[db]
            return carry

        lax.fori_loop(0, HALF, n_body, 0, unroll=2)

    def start_dma(r, h, slot):
        return pltpu.async_copy(
            bufs[slot], out_hbm.at[base + r, pl.ds(h * HALF, HALF)],
            sems[slot])

    for h in range(2):
        pltpu.sync_copy(emb_s_hbm.at[pl.ds(h * HALF, HALF)], s_v)
        compute_row(0, 0)
        start_dma(0, h, 0)
        compute_row(1, 1)
        start_dma(1, h, 1)

        def pair_body(i, carry):
            for b in range(2):
                r = 2 * i + b
                pltpu.make_async_copy(
                    bufs[b], out_hbm.at[base + r, pl.ds(h * HALF, HALF)],
                    sems[b]).wait()
                compute_row(r, b)
                start_dma(r, h, b)
            return carry

        lax.fori_loop(1, RPW // 2, pair_body, 0)
        for b in range(2):
            pltpu.make_async_copy(
                bufs[b], out_hbm.at[base, pl.ds(h * HALF, HALF)],
                sems[b]).wait()


_sc_writer = functools.partial(
    pl.kernel,
    out_type=jax.ShapeDtypeStruct((BT, NUM_NODES, MODEL_DIM), jnp.float32),
    mesh=plsc.VectorSubcoreMesh(core_axis_name="c", subcore_axis_name="s"),
    scratch_types=[
        pltpu.VMEM((HALF, MODEL_DIM), jnp.float32),
        pltpu.VMEM((RPW + 4, MODEL_DIM), jnp.float32),
        pltpu.VMEM((2, HALF, MODEL_DIM), jnp.float32),
        pltpu.SemaphoreType.DMA,
        pltpu.SemaphoreType.DMA,
    ],
)(_sc_writer_body)


def kernel(time, weekday, node_table, W_node, time_table, W_time,
           weekday_table, W_weekday):
    t_flat = time.reshape(BT, 1).astype(jnp.int32)
    w_flat = weekday.reshape(BT, 1).astype(jnp.int32)
    wdt_pad = jnp.pad(weekday_table, ((0, 1), (0, 0)))

    emb_s, emb_t = _prep(t_flat, w_flat, node_table, W_node, time_table,
                         W_time, wdt_pad, W_weekday)
    out = _sc_writer(emb_s, emb_t)
    return out.reshape(B, T, NUM_NODES, MODEL_DIM)


# speedup vs baseline: 2.8038x; 2.8038x over previous
import functools

import jax
import jax.numpy as jnp
from jax import lax
from jax.experimental import pallas as pl
from jax.experimental.pallas import tpu as pltpu
from jax.experimental.pallas import tpu_sc as plsc

NUM_NODES = 512
NODE_DIM = 64
NUM_TIMES = 288
TIME_DIM = 32
WEEKDAY_DIM = 16
MODEL_DIM = 128
B, T = 32, 12
BT = B * T
LANES = 16
DBLK = MODEL_DIM // LANES

ROWS_PER_STEP = 16
GW = 24
GR = BT // GW


def _prep_body(tt_ref, wt_ref, wdt_ref, ww_ref, pt_ref, pw_ref):
    pt_ref[...] = jnp.dot(tt_ref[...], wt_ref[...],
                          preferred_element_type=jnp.float32)
    pw_ref[...] = jnp.dot(wdt_ref[...], ww_ref[...],
                          preferred_element_type=jnp.float32)


_prep = pl.pallas_call(
    _prep_body,
    out_shape=(
        jax.ShapeDtypeStruct((NUM_TIMES, MODEL_DIM), jnp.float32),
        jax.ShapeDtypeStruct((8, MODEL_DIM), jnp.float32),
    ),
)


def _gather_body(tidx_hbm, widx_hbm, pt_hbm, pw_hbm, out_hbm,
                 tidx_v, widx_v, trows_v, wrows_v, sem_t, sem_w):
    wid = lax.axis_index("s") * 2 + lax.axis_index("c")

    @pl.when(wid < GW)
    def _():
        base = pl.multiple_of(wid * GR, 8)
        pltpu.sync_copy(tidx_hbm.at[pl.ds(base, GR)], tidx_v)
        pltpu.sync_copy(widx_hbm.at[pl.ds(base, GR)], widx_v)
        ct = pltpu.async_copy(pt_hbm.at[tidx_v], trows_v, sem_t)
        cw = pltpu.async_copy(pw_hbm.at[widx_v], wrows_v, sem_w)
        ct.wait()
        cw.wait()
        for r in range(GR):
            for db in range(DBLK):
                sl = pl.ds(db * LANES, LANES)
                trows_v[r, sl] = trows_v[r, sl] + wrows_v[r, sl]
        pltpu.sync_copy(trows_v, out_hbm.at[pl.ds(base, GR)])


_gather = functools.partial(
    pl.kernel,
    out_type=jax.ShapeDtypeStruct((BT, MODEL_DIM), jnp.float32),
    mesh=plsc.VectorSubcoreMesh(core_axis_name="c", subcore_axis_name="s"),
    scratch_types=[
        pltpu.VMEM((GR,), jnp.int32),
        pltpu.VMEM((GR,), jnp.int32),
        pltpu.VMEM((GR, MODEL_DIM), jnp.float32),
        pltpu.VMEM((GR, MODEL_DIM), jnp.float32),
        pltpu.SemaphoreType.DMA,
        pltpu.SemaphoreType.DMA,
    ],
)(_gather_body)


def _writer_body(node_ref, wn_ref, emb_t_ref, out_ref, emb_s_ref):
    i = pl.program_id(0)

    @pl.when(i == 0)
    def _init():
        emb_s_ref[...] = jnp.dot(node_ref[...], wn_ref[...],
                                 preferred_element_type=jnp.float32)

    rows = emb_t_ref[pl.ds(i * ROWS_PER_STEP, ROWS_PER_STEP), :]
    out_ref[...] = emb_s_ref[...][None, :, :] + rows[:, None, :]


_full = lambda shape: pl.BlockSpec(shape, lambda i: (0,) * len(shape))
_writer = pl.pallas_call(
    _writer_body,
    grid=(BT // ROWS_PER_STEP,),
    in_specs=[
        _full((NUM_NODES, NODE_DIM)),
        _full((NODE_DIM, MODEL_DIM)),
        _full((BT, MODEL_DIM)),
    ],
    out_specs=pl.BlockSpec((ROWS_PER_STEP, NUM_NODES, MODEL_DIM),
                           lambda i: (i, 0, 0)),
    out_shape=jax.ShapeDtypeStruct((BT, NUM_NODES, MODEL_DIM), jnp.float32),
    scratch_shapes=[pltpu.VMEM((NUM_NODES, MODEL_DIM), jnp.float32)],
)


def kernel(time, weekday, node_table, W_node, time_table, W_time,
           weekday_table, W_weekday):
    t_flat = time.reshape(BT).astype(jnp.int32)
    w_flat = weekday.reshape(BT).astype(jnp.int32)
    wdt_pad = jnp.pad(weekday_table, ((0, 1), (0, 0)))

    pt, pw = _prep(time_table, W_time, wdt_pad, W_weekday)
    emb_t = _gather(t_flat, w_flat, pt, pw)
    out = _writer(node_table, W_node, emb_t)
    return out.reshape(B, T, NUM_NODES, MODEL_DIM)


# speedup vs baseline: 4.1055x; 1.4643x over previous
import functools

import jax
import jax.numpy as jnp
from jax.experimental import pallas as pl
from jax.experimental.pallas import tpu as pltpu

NUM_NODES = 512
NODE_DIM = 64
NUM_TIMES = 288
TIME_DIM = 32
WEEKDAY_DIM = 16
MODEL_DIM = 128
B, T = 32, 12
BT = B * T
ROWS_PER_STEP = 32


def _body(time_ref, wd_ref, node_ref, wn_ref, tt_ref, wt_ref, wdt_ref, ww_ref,
          out_ref, emb_s_ref, emb_t_ref, *, row_base, n_rows):
    i = pl.program_id(0)

    @pl.when(i == 0)
    def _init():
        emb_s_ref[...] = jnp.dot(node_ref[...], wn_ref[...],
                                 preferred_element_type=jnp.float32)
        t_idx = time_ref[...]
        w_idx = wd_ref[...]
        t_iota = jax.lax.broadcasted_iota(jnp.int32, (n_rows, NUM_TIMES), 1)
        w_iota = jax.lax.broadcasted_iota(jnp.int32, (n_rows, 8), 1)
        t_oh = (t_idx == t_iota).astype(jnp.float32)
        w_oh = (w_idx == w_iota).astype(jnp.float32)
        g_t = jnp.dot(t_oh, tt_ref[...], preferred_element_type=jnp.float32)
        g_w = jnp.dot(w_oh, wdt_ref[...], preferred_element_type=jnp.float32)
        emb_t_ref[...] = (
            jnp.dot(g_t, wt_ref[...], preferred_element_type=jnp.float32)
            + jnp.dot(g_w, ww_ref[...], preferred_element_type=jnp.float32))

    rows = emb_t_ref[pl.ds(i * ROWS_PER_STEP, ROWS_PER_STEP), :]
    out_ref[...] = emb_s_ref[...][None, :, :] + rows[:, None, :]


def _make_writer(n_rows, row_base):
    full = lambda shape: pl.BlockSpec(shape, lambda i: (0,) * len(shape))
    return pl.pallas_call(
        functools.partial(_body, row_base=row_base, n_rows=n_rows),
        grid=(n_rows // ROWS_PER_STEP,),
        in_specs=[
            full((n_rows, 1)),
            full((n_rows, 1)),
            full((NUM_NODES, NODE_DIM)),
            full((NODE_DIM, MODEL_DIM)),
            full((NUM_TIMES, TIME_DIM)),
            full((TIME_DIM, MODEL_DIM)),
            full((8, WEEKDAY_DIM)),
            full((WEEKDAY_DIM, MODEL_DIM)),
        ],
        out_specs=pl.BlockSpec((ROWS_PER_STEP, NUM_NODES, MODEL_DIM),
                               lambda i: (i, 0, 0)),
        out_shape=jax.ShapeDtypeStruct((n_rows, NUM_NODES, MODEL_DIM),
                                       jnp.float32),
        scratch_shapes=[
            pltpu.VMEM((NUM_NODES, MODEL_DIM), jnp.float32),
            pltpu.VMEM((n_rows, MODEL_DIM), jnp.float32),
        ],
    )


def kernel(time, weekday, node_table, W_node, time_table, W_time,
           weekday_table, W_weekday):
    t_flat = time.reshape(BT, 1).astype(jnp.int32)
    w_flat = weekday.reshape(BT, 1).astype(jnp.int32)
    wdt_pad = jnp.pad(weekday_table, ((0, 1), (0, 0)))

    tabs = (node_table, W_node, time_table, W_time, wdt_pad, W_weekday)
    out = _make_writer(BT, 0)(t_flat, w_flat, *tabs)
    return out.reshape(B, T, NUM_NODES, MODEL_DIM)


# speedup vs baseline: 4.3452x; 1.0584x over previous
import jax
import jax.numpy as jnp
from jax import lax
from jax.experimental import pallas as pl
from jax.experimental.pallas import tpu as pltpu

NUM_NODES = 512
NODE_DIM = 64
NUM_TIMES = 288
TIME_DIM = 32
WEEKDAY_DIM = 16
MODEL_DIM = 128
B, T = 32, 12


def _body(time_ref, wd_ref, node_ref, wn_ref, tt_ref, wt_ref, wdt_ref, ww_ref,
          out_ref, emb_s_ref, emb_t_ref):
    b = pl.program_id(0)

    @pl.when(b == 0)
    def _init():
        emb_s_ref[...] = jnp.dot(node_ref[...], wn_ref[...],
                                 preferred_element_type=jnp.float32)
        pt = jnp.dot(tt_ref[...], wt_ref[...],
                     preferred_element_type=jnp.float32)
        wdt_pad = jnp.concatenate(
            [wdt_ref[...], jnp.zeros((1, WEEKDAY_DIM), jnp.float32)], axis=0)
        pw = jnp.dot(wdt_pad, ww_ref[...],
                     preferred_element_type=jnp.float32)
        t_iota = lax.broadcasted_iota(jnp.int32, (B, NUM_TIMES), 1)
        w_iota = lax.broadcasted_iota(jnp.int32, (B, 8), 1)
        for t in range(T):
            t_oh = (time_ref[:, t:t + 1] == t_iota).astype(jnp.float32)
            w_oh = (wd_ref[:, t:t + 1] == w_iota).astype(jnp.float32)
            emb_t_ref[:, t, :] = (
                jnp.dot(t_oh, pt, preferred_element_type=jnp.float32)
                + jnp.dot(w_oh, pw, preferred_element_type=jnp.float32))

    rows = emb_t_ref[pl.ds(b, 1)]
    out_ref[...] = emb_s_ref[...][None, None, :, :] + rows[:, :, None, :]


_full = lambda shape: pl.BlockSpec(shape, lambda i: (0,) * len(shape))


def kernel(time, weekday, node_table, W_node, time_table, W_time,
           weekday_table, W_weekday):
    out = pl.pallas_call(
        _body,
        grid=(B,),
        in_specs=[
            _full((B, T)),
            _full((B, T)),
            _full((NUM_NODES, NODE_DIM)),
            _full((NODE_DIM, MODEL_DIM)),
            _full((NUM_TIMES, TIME_DIM)),
            _full((TIME_DIM, MODEL_DIM)),
            _full((7, WEEKDAY_DIM)),
            _full((WEEKDAY_DIM, MODEL_DIM)),
        ],
        out_specs=pl.BlockSpec((1, T, NUM_NODES, MODEL_DIM),
                               lambda i: (i, 0, 0, 0)),
        out_shape=jax.ShapeDtypeStruct((B, T, NUM_NODES, MODEL_DIM),
                                       jnp.float32),
        scratch_shapes=[
            pltpu.VMEM((NUM_NODES, MODEL_DIM), jnp.float32),
            pltpu.VMEM((B, T, MODEL_DIM), jnp.float32),
        ],
    )(time, weekday, node_table, W_node, time_table, W_time,
      weekday_table, W_weekday)
    return out


# speedup vs baseline: 4.4082x; 1.0145x over previous
import jax
import jax.numpy as jnp
from jax import lax
from jax.experimental import pallas as pl
from jax.experimental.pallas import tpu as pltpu

NUM_NODES = 512
NODE_DIM = 64
NUM_TIMES = 288
TIME_DIM = 32
WEEKDAY_DIM = 16
MODEL_DIM = 128
B, T = 32, 12
BT = B * T
ROWS_PER_STEP = 16


def _flatten_idx(idx_ref):
    r_row = lax.broadcasted_iota(jnp.int32, (BT, B), 0) // T
    b_col = lax.broadcasted_iota(jnp.int32, (BT, B), 1)
    rowsel = (r_row == b_col).astype(jnp.float32)
    picked = jnp.dot(rowsel, idx_ref[...].astype(jnp.float32),
                     preferred_element_type=jnp.float32)
    r_mod = lax.broadcasted_iota(jnp.int32, (BT, T), 0) % T
    t_col = lax.broadcasted_iota(jnp.int32, (BT, T), 1)
    colmask = (r_mod == t_col).astype(jnp.float32)
    return jnp.sum(picked * colmask, axis=1, keepdims=True)


def _body(time_ref, wd_ref, node_ref, wn_ref, tt_ref, wt_ref, wdt_ref, ww_ref,
          out_ref, emb_s_ref, emb_t_ref):
    i = pl.program_id(0)

    @pl.when(i == 0)
    def _init():
        emb_s_ref[...] = jnp.dot(node_ref[...], wn_ref[...],
                                 preferred_element_type=jnp.float32)
        t_idx = _flatten_idx(time_ref).astype(jnp.int32)
        w_idx = _flatten_idx(wd_ref).astype(jnp.int32)
        t_iota = lax.broadcasted_iota(jnp.int32, (BT, NUM_TIMES), 1)
        w_iota = lax.broadcasted_iota(jnp.int32, (BT, 8), 1)
        t_oh = (t_idx == t_iota).astype(jnp.float32)
        w_oh = (w_idx == w_iota).astype(jnp.float32)
        wdt_pad = jnp.concatenate(
            [wdt_ref[...], jnp.zeros((1, WEEKDAY_DIM), jnp.float32)], axis=0)
        g_t = jnp.dot(t_oh, tt_ref[...], preferred_element_type=jnp.float32)
        g_w = jnp.dot(w_oh, wdt_pad, preferred_element_type=jnp.float32)
        emb_t_ref[...] = (
            jnp.dot(g_t, wt_ref[...], preferred_element_type=jnp.float32)
            + jnp.dot(g_w, ww_ref[...], preferred_element_type=jnp.float32))

    rows = emb_t_ref[pl.ds(i * ROWS_PER_STEP, ROWS_PER_STEP), :]
    out_ref[...] = emb_s_ref[...][None, :, :] + rows[:, None, :]


_full = lambda shape: pl.BlockSpec(shape, lambda i: (0,) * len(shape))


def kernel(time, weekday, node_table, W_node, time_table, W_time,
           weekday_table, W_weekday):
    out = pl.pallas_call(
        _body,
        grid=(BT // ROWS_PER_STEP,),
        in_specs=[
            _full((B, T)),
            _full((B, T)),
            _full((NUM_NODES, NODE_DIM)),
            _full((NODE_DIM, MODEL_DIM)),
            _full((NUM_TIMES, TIME_DIM)),
            _full((TIME_DIM, MODEL_DIM)),
            _full((7, WEEKDAY_DIM)),
            _full((WEEKDAY_DIM, MODEL_DIM)),
        ],
        out_specs=pl.BlockSpec((ROWS_PER_STEP, NUM_NODES, MODEL_DIM),
                               lambda i: (i, 0, 0)),
        out_shape=jax.ShapeDtypeStruct((BT, NUM_NODES, MODEL_DIM),
                                       jnp.float32),
        scratch_shapes=[
            pltpu.VMEM((NUM_NODES, MODEL_DIM), jnp.float32),
            pltpu.VMEM((BT, MODEL_DIM), jnp.float32),
        ],
    )(time, weekday, node_table, W_node, time_table, W_time,
      weekday_table, W_weekday)
    return out.reshape(B, T, NUM_NODES, MODEL_DIM)
